# baseline (speedup 1.0000x reference)
import jax
import jax.numpy as jnp
from jax import lax
from jax.experimental import pallas as pl
from jax.experimental.pallas import tpu as pltpu
from jax.experimental.pallas import tpu_sc as plsc

DIM = 1024
B, S = 4, 2048
TOK = B * S
EPS = 1e-5
LANES = 16
J = DIM // LANES

NC, NS = 2, 16
NW = NC * NS
SPW = S // NW
SC_ = 4
C = SC_ * B
NCHUNK = SPW // SC_
JB = 8


def _rsqrt_newton(x):
    i = lax.bitcast_convert_type(x, jnp.int32)
    i = jnp.int32(0x5F3759DF) - lax.shift_right_arithmetic(i, 1)
    y = lax.bitcast_convert_type(i, jnp.float32)
    for _ in range(3):
        y = y * (1.5 - 0.5 * x * y * y)
    return y


def _emb_ln_body(ids_hbm, pos_hbm, gamma_hbm, beta_hbm, table_hbm, out_hbm,
                 ids_v, wbufs, obufs, pbufs, g_v, b_v, r_s, m_s,
                 gsems, psems, ssems):
    wid = lax.axis_index("s") * NC + lax.axis_index("c")
    s_lo = wid * SPW

    pltpu.sync_copy(ids_hbm.at[pl.ds(wid * (B * SPW), B * SPW)], ids_v)
    pltpu.sync_copy(gamma_hbm, g_v)
    pltpu.sync_copy(beta_hbm, b_v)

    def issue_gather(k):
        idx = ids_v.at[pl.ds(k * C, C)]
        return pltpu.async_copy(table_hbm.at[idx], wbufs[k % 2], gsems[k % 2])

    def issue_pos(k):
        return pltpu.async_copy(pos_hbm.at[pl.ds(s_lo + k * SC_, SC_)],
                                pbufs[k % 2], psems[k % 2])

    def issue_stores(k):
        obuf = obufs[k % 2]
        sem = ssems[k % 2]
        descs = []
        for b in range(B):
            dst = out_hbm.at[pl.ds(b * S + s_lo + k * SC_, SC_)]
            descs.append(pltpu.async_copy(obuf.at[pl.ds(b * SC_, SC_)],
                                          dst, sem))
        return descs

    def compute(k):
        wbuf = wbufs[k % 2]
        obuf = obufs[k % 2]
        pbuf = pbufs[k % 2]

        @plsc.parallel_loop(0, SC_)
        def _si_stats(si):
            zeros = tuple(jnp.zeros((LANES,), jnp.float32) for _ in range(8))

            def jbody(j, accs):
                accs = list(accs)
                sl = pl.ds(j * LANES, LANES)
                p = pbuf[si, sl]
                for b in range(B):
                    x = wbuf[b * SC_ + si, sl] + p
                    obuf[b * SC_ + si, sl] = x
                    accs[2 * b] = accs[2 * b] + x
                    accs[2 * b + 1] = accs[2 * b + 1] + x * x
                return tuple(accs)

            accs = plsc.parallel_loop(0, J, unroll=8, carry=zeros)(jbody)
            for b in range(B):
                mu = jnp.sum(accs[2 * b]) * (1.0 / DIM)
                var = jnp.sum(accs[2 * b + 1]) * (1.0 / DIM) - mu * mu
                r = _rsqrt_newton(var + EPS)
                r_s[b * SC_ + si] = r
                m_s[b * SC_ + si] = mu

        @plsc.parallel_loop(0, J // JB)
        def _col_apply(jb):
            j0 = jb * JB
            gs = [g_v[pl.ds((j0 + u) * LANES, LANES)] for u in range(JB)]
            bs = [b_v[pl.ds((j0 + u) * LANES, LANES)] for u in range(JB)]

            @plsc.parallel_loop(0, C)
            def _tok_apply(t):
                rt = r_s[t]
                mt = m_s[t]
                for u in range(JB):
                    sl = pl.ds((j0 + u) * LANES, LANES)
                    obuf[t, sl] = ((obuf[t, sl] - mt) * rt) * gs[u] + bs[u]

    gather_descs = [issue_gather(0), issue_gather(1)]
    pos_descs = [issue_pos(0), issue_pos(1)]
    store_descs = []
    for k in range(NCHUNK):
        gather_descs[k].wait()
        pos_descs[k].wait()
        if k >= 2:
            for d in store_descs[k - 2]:
                d.wait()
        compute(k)
        store_descs.append(issue_stores(k))
        if k + 2 < NCHUNK:
            gather_descs.append(issue_gather(k + 2))
            pos_descs.append(issue_pos(k + 2))
    for k in (NCHUNK - 2, NCHUNK - 1):
        for d in store_descs[k]:
            d.wait()


@jax.jit
def kernel(input_ids, x_qkv, word_table, pos_table, gamma, beta):
    del x_qkv
    ids_r = (input_ids.astype(jnp.int32)
             .reshape(B, NW, NCHUNK, SC_)
             .transpose(1, 2, 0, 3)
             .reshape(TOK))

    mesh = plsc.VectorSubcoreMesh(
        core_axis_name="c", subcore_axis_name="s",
        num_cores=NC, num_subcores=NS)

    run = pl.kernel(
        _emb_ln_body,
        out_type=jax.ShapeDtypeStruct((TOK, DIM), jnp.float32),
        mesh=mesh,
        compiler_params=pltpu.CompilerParams(needs_layout_passes=False),
        scratch_types=[
            pltpu.VMEM((B * SPW,), jnp.int32),
            [pltpu.VMEM((C, DIM), jnp.float32) for _ in range(2)],
            [pltpu.VMEM((C, DIM), jnp.float32) for _ in range(2)],
            [pltpu.VMEM((SC_, DIM), jnp.float32) for _ in range(2)],
            pltpu.VMEM((DIM,), jnp.float32),
            pltpu.VMEM((DIM,), jnp.float32),
            pltpu.SMEM((C,), jnp.float32),
            pltpu.SMEM((C,), jnp.float32),
            [pltpu.SemaphoreType.DMA for _ in range(2)],
            [pltpu.SemaphoreType.DMA for _ in range(2)],
            [pltpu.SemaphoreType.DMA for _ in range(2)],
        ],
    )
    out = run(ids_r, pos_table, gamma, beta, word_table)
    return out.reshape(B, S, DIM)


# speedup vs baseline: 1.0108x; 1.0108x over previous
import jax
import jax.numpy as jnp
from jax import lax
from jax.experimental import pallas as pl
from jax.experimental.pallas import tpu as pltpu
from jax.experimental.pallas import tpu_sc as plsc

DIM = 1024
B, S = 4, 2048
TOK = B * S
EPS = 1e-5
LANES = 16
J = DIM // LANES

NC, NS = 2, 16
NW = NC * NS
SPW = S // NW
SC_ = 4
C = SC_ * B
NCHUNK = SPW // SC_
JB = 16


def _rsqrt_newton(x):
    i = lax.bitcast_convert_type(x, jnp.int32)
    i = jnp.int32(0x5F3759DF) - lax.shift_right_arithmetic(i, 1)
    y = lax.bitcast_convert_type(i, jnp.float32)
    for _ in range(3):
        y = y * (1.5 - 0.5 * x * y * y)
    return y


def _emb_ln_body(ids_hbm, pos_hbm, gamma_hbm, beta_hbm, table_hbm, out_hbm,
                 ids_v, wbufs, obufs, pbufs, g_v, b_v, r_s, m_s,
                 gsems, psems, ssems):
    wid = lax.axis_index("s") * NC + lax.axis_index("c")
    s_lo = wid * SPW

    pltpu.sync_copy(ids_hbm.at[pl.ds(wid * (B * SPW), B * SPW)], ids_v)
    pltpu.sync_copy(gamma_hbm, g_v)
    pltpu.sync_copy(beta_hbm, b_v)

    def issue_gather(k):
        idx = ids_v.at[pl.ds(k * C, C)]
        return pltpu.async_copy(table_hbm.at[idx], wbufs[k % 2], gsems[k % 2])

    def issue_pos(k):
        return pltpu.async_copy(pos_hbm.at[pl.ds(s_lo + k * SC_, SC_)],
                                pbufs[k % 2], psems[k % 2])

    def issue_stores(k):
        obuf = obufs[k % 2]
        sem = ssems[k % 2]
        descs = []
        for b in range(B):
            dst = out_hbm.at[pl.ds(b * S + s_lo + k * SC_, SC_)]
            descs.append(pltpu.async_copy(obuf.at[pl.ds(b * SC_, SC_)],
                                          dst, sem))
        return descs

    def compute(k):
        wbuf = wbufs[k % 2]
        obuf = obufs[k % 2]
        pbuf = pbufs[k % 2]

        @plsc.parallel_loop(0, SC_)
        def _si_stats(si):
            zeros = tuple(jnp.zeros((LANES,), jnp.float32) for _ in range(8))

            def jbody(j, accs):
                accs = list(accs)
                sl = pl.ds(j * LANES, LANES)
                p = pbuf[si, sl]
                for b in range(B):
                    x = wbuf[b * SC_ + si, sl] + p
                    obuf[b * SC_ + si, sl] = x
                    accs[2 * b] = accs[2 * b] + x
                    accs[2 * b + 1] = accs[2 * b + 1] + x * x
                return tuple(accs)

            accs = plsc.parallel_loop(0, J, unroll=4, carry=zeros)(jbody)
            for b in range(B):
                mu = jnp.sum(accs[2 * b]) * (1.0 / DIM)
                var = jnp.sum(accs[2 * b + 1]) * (1.0 / DIM) - mu * mu
                r = _rsqrt_newton(var + EPS)
                r_s[b * SC_ + si] = r
                m_s[b * SC_ + si] = mu

        @plsc.parallel_loop(0, J // JB)
        def _col_apply(jb):
            j0 = jb * JB
            gs = [g_v[pl.ds((j0 + u) * LANES, LANES)] for u in range(JB)]
            bs = [b_v[pl.ds((j0 + u) * LANES, LANES)] for u in range(JB)]

            @plsc.parallel_loop(0, C)
            def _tok_apply(t):
                rt = r_s[t]
                mt = m_s[t]
                for u in range(JB):
                    sl = pl.ds((j0 + u) * LANES, LANES)
                    obuf[t, sl] = ((obuf[t, sl] - mt) * rt) * gs[u] + bs[u]

    gather_descs = [issue_gather(0), issue_gather(1)]
    pos_descs = [issue_pos(0), issue_pos(1)]
    store_descs = []
    for k in range(NCHUNK):
        gather_descs[k].wait()
        pos_descs[k].wait()
        if k >= 2:
            for d in store_descs[k - 2]:
                d.wait()
        compute(k)
        store_descs.append(issue_stores(k))
        if k + 2 < NCHUNK:
            gather_descs.append(issue_gather(k + 2))
            pos_descs.append(issue_pos(k + 2))
    for k in (NCHUNK - 2, NCHUNK - 1):
        for d in store_descs[k]:
            d.wait()


@jax.jit
def kernel(input_ids, x_qkv, word_table, pos_table, gamma, beta):
    del x_qkv
    ids_r = (input_ids.astype(jnp.int32)
             .reshape(B, NW, NCHUNK, SC_)
             .transpose(1, 2, 0, 3)
             .reshape(TOK))

    mesh = plsc.VectorSubcoreMesh(
        core_axis_name="c", subcore_axis_name="s",
        num_cores=NC, num_subcores=NS)

    run = pl.kernel(
        _emb_ln_body,
        out_type=jax.ShapeDtypeStruct((TOK, DIM), jnp.float32),
        mesh=mesh,
        compiler_params=pltpu.CompilerParams(needs_layout_passes=False),
        scratch_types=[
            pltpu.VMEM((B * SPW,), jnp.int32),
            [pltpu.VMEM((C, DIM), jnp.float32) for _ in range(2)],
            [pltpu.VMEM((C, DIM), jnp.float32) for _ in range(2)],
            [pltpu.VMEM((SC_, DIM), jnp.float32) for _ in range(2)],
            pltpu.VMEM((DIM,), jnp.float32),
            pltpu.VMEM((DIM,), jnp.float32),
            pltpu.SMEM((C,), jnp.float32),
            pltpu.SMEM((C,), jnp.float32),
            [pltpu.SemaphoreType.DMA for _ in range(2)],
            [pltpu.SemaphoreType.DMA for _ in range(2)],
            [pltpu.SemaphoreType.DMA for _ in range(2)],
        ],
    )
    out = run(ids_r, pos_table, gamma, beta, word_table)
    return out.reshape(B, S, DIM)


# speedup vs baseline: 1.0379x; 1.0268x over previous
import jax
import jax.numpy as jnp
from jax import lax
from jax.experimental import pallas as pl
from jax.experimental.pallas import tpu as pltpu
from jax.experimental.pallas import tpu_sc as plsc

DIM = 1024
B, S = 4, 2048
TOK = B * S
EPS = 1e-5
LANES = 16
J = DIM // LANES

NC, NS = 2, 16
NW = NC * NS
SPW = S // NW
SC_ = 4
C = SC_ * B
NCHUNK = SPW // SC_
JB = 16


def _rsqrt_newton(x):
    i = lax.bitcast_convert_type(x, jnp.int32)
    i = jnp.int32(0x5F3759DF) - lax.shift_right_arithmetic(i, 1)
    y = lax.bitcast_convert_type(i, jnp.float32)
    for _ in range(3):
        y = y * (1.5 - 0.5 * x * y * y)
    return y


def _emb_ln_body(ids_hbm, pos_hbm, gamma_hbm, beta_hbm, table_hbm, out_hbm,
                 ids_v, wbufs, obufs, pbufs, g_v, b_v, r_s, m_s,
                 gsems, psems, ssems):
    wid = lax.axis_index("s") * NC + lax.axis_index("c")
    s_lo = wid * SPW

    pltpu.sync_copy(ids_hbm.at[pl.ds(wid * (B * SPW), B * SPW)], ids_v)
    pltpu.sync_copy(gamma_hbm, g_v)
    pltpu.sync_copy(beta_hbm, b_v)

    def issue_gather(k):
        idx = ids_v.at[pl.ds(k * C, C)]
        return pltpu.async_copy(table_hbm.at[idx], wbufs[k % 2], gsems[k % 2])

    def issue_pos(k):
        return pltpu.async_copy(pos_hbm.at[pl.ds(s_lo + k * SC_, SC_)],
                                pbufs[k % 2], psems[k % 2])

    def issue_stores(k):
        dst = out_hbm.at[:, pl.ds(s_lo + k * SC_, SC_), :]
        return [pltpu.async_copy(obufs[k % 2], dst, ssems[k % 2])]

    def compute(k):
        wbuf = wbufs[k % 2]
        obuf = obufs[k % 2]
        pbuf = pbufs[k % 2]

        @plsc.parallel_loop(0, SC_)
        def _si_stats(si):
            zeros = tuple(jnp.zeros((LANES,), jnp.float32) for _ in range(8))

            def jbody(j, accs):
                accs = list(accs)
                sl = pl.ds(j * LANES, LANES)
                p = pbuf[si, sl]
                for b in range(B):
                    x = wbuf[b * SC_ + si, sl] + p
                    obuf[b, si, sl] = x
                    accs[2 * b] = accs[2 * b] + x
                    accs[2 * b + 1] = accs[2 * b + 1] + x * x
                return tuple(accs)

            accs = plsc.parallel_loop(0, J, unroll=4, carry=zeros)(jbody)
            for b in range(B):
                mu = jnp.sum(accs[2 * b]) * (1.0 / DIM)
                var = jnp.sum(accs[2 * b + 1]) * (1.0 / DIM) - mu * mu
                r = _rsqrt_newton(var + EPS)
                r_s[b * SC_ + si] = r
                m_s[b * SC_ + si] = mu

        @plsc.parallel_loop(0, J // JB)
        def _col_apply(jb):
            j0 = jb * JB
            gs = [g_v[pl.ds((j0 + u) * LANES, LANES)] for u in range(JB)]
            bs = [b_v[pl.ds((j0 + u) * LANES, LANES)] for u in range(JB)]

            @plsc.parallel_loop(0, C)
            def _tok_apply(t):
                rt = r_s[t]
                mt = m_s[t]
                bi = lax.shift_right_logical(t, 2)
                si = lax.bitwise_and(t, SC_ - 1)
                for u in range(JB):
                    sl = pl.ds((j0 + u) * LANES, LANES)
                    obuf[bi, si, sl] = (((obuf[bi, si, sl] - mt) * rt)
                                        * gs[u] + bs[u])

    gather_descs = [issue_gather(0), issue_gather(1)]
    pos_descs = [issue_pos(0), issue_pos(1)]
    store_descs = []
    for k in range(NCHUNK):
        gather_descs[k].wait()
        pos_descs[k].wait()
        if k >= 2:
            for d in store_descs[k - 2]:
                d.wait()
        compute(k)
        store_descs.append(issue_stores(k))
        if k + 2 < NCHUNK:
            gather_descs.append(issue_gather(k + 2))
            pos_descs.append(issue_pos(k + 2))
    for k in (NCHUNK - 2, NCHUNK - 1):
        for d in store_descs[k]:
            d.wait()


@jax.jit
def kernel(input_ids, x_qkv, word_table, pos_table, gamma, beta):
    del x_qkv
    ids_r = (input_ids.astype(jnp.int32)
             .reshape(B, NW, NCHUNK, SC_)
             .transpose(1, 2, 0, 3)
             .reshape(TOK))

    mesh = plsc.VectorSubcoreMesh(
        core_axis_name="c", subcore_axis_name="s",
        num_cores=NC, num_subcores=NS)

    run = pl.kernel(
        _emb_ln_body,
        out_type=jax.ShapeDtypeStruct((B, S, DIM), jnp.float32),
        mesh=mesh,
        compiler_params=pltpu.CompilerParams(needs_layout_passes=False),
        scratch_types=[
            pltpu.VMEM((B * SPW,), jnp.int32),
            [pltpu.VMEM((C, DIM), jnp.float32) for _ in range(2)],
            [pltpu.VMEM((B, SC_, DIM), jnp.float32) for _ in range(2)],
            [pltpu.VMEM((SC_, DIM), jnp.float32) for _ in range(2)],
            pltpu.VMEM((DIM,), jnp.float32),
            pltpu.VMEM((DIM,), jnp.float32),
            pltpu.SMEM((C,), jnp.float32),
            pltpu.SMEM((C,), jnp.float32),
            [pltpu.SemaphoreType.DMA for _ in range(2)],
            [pltpu.SemaphoreType.DMA for _ in range(2)],
            [pltpu.SemaphoreType.DMA for _ in range(2)],
        ],
    )
    return run(ids_r, pos_table, gamma, beta, word_table)
